# Initial kernel scaffold; baseline (speedup 1.0000x reference)
import jax
import jax.numpy as jnp
from jax.experimental import pallas as pl


def kernel(feat, edge_index, efeat, W_edge, b_edge, W_node, b_node, Wn_ih, Wn_hh, bn_ih, bn_hh, We_ih, We_hh, be_ih, be_hh, W1, b1, W2, b2, W3, b3):
    raise NotImplementedError("write your pallas kernel here")


# speedup vs baseline: 1.6345x; 1.6345x over previous
import jax
import jax.numpy as jnp
from jax import lax
from jax.experimental import pallas as pl
from jax.experimental.pallas import tpu as pltpu
from jax.experimental.pallas import tpu_sc as plsc

N = 10000
E = 320000
D = 128
DE = 16
NS = 16
NP = 10240
SROWS = 10112
SPAD = SROWS - 1
CH = 32
CPG = 8
GROUPS = 79
PER_TILE = GROUPS * CPG * CH
E_PAD = NS * PER_TILE
ROWS_PER_TILE = 624
ROWS_REM = N - NS * ROWS_PER_TILE
ZROWS = SROWS // NS
DEG_COLS = NP // NS
NBLK = 10
BLK = N // NBLK
EBLK = 1024


def _ab_body(feat_ref, ws_ref, wd_ref, a_ref, b_ref):
    f = feat_ref[...]
    a_ref[...] = jnp.dot(f, ws_ref[...], preferred_element_type=jnp.float32)
    b_ref[...] = jnp.dot(f, wd_ref[...], preferred_element_type=jnp.float32)


def _edge_tables_ab(feat, ws, wd):
    return pl.pallas_call(
        _ab_body,
        grid=(NBLK,),
        in_specs=[
            pl.BlockSpec((BLK, D), lambda i: (i, 0)),
            pl.BlockSpec((D, D), lambda i: (0, 0)),
            pl.BlockSpec((D, D), lambda i: (0, 0)),
        ],
        out_specs=[
            pl.BlockSpec((BLK, D), lambda i: (i, 0)),
            pl.BlockSpec((BLK, D), lambda i: (i, 0)),
        ],
        out_shape=[
            jax.ShapeDtypeStruct((N, D), jnp.float32),
            jax.ShapeDtypeStruct((N, D), jnp.float32),
        ],
    )(feat, ws, wd)


def _c_body(ef_ref, we_ref, be_ref, c_ref):
    c_ref[...] = (
        jnp.dot(ef_ref[...], we_ref[...], preferred_element_type=jnp.float32)
        + be_ref[...]
    )


def _edge_table_c(efeat_p, we, be):
    return pl.pallas_call(
        _c_body,
        grid=(E_PAD // EBLK,),
        in_specs=[
            pl.BlockSpec((EBLK, DE), lambda i: (i, 0)),
            pl.BlockSpec((DE, D), lambda i: (0, 0)),
            pl.BlockSpec((1, D), lambda i: (0, 0)),
        ],
        out_specs=pl.BlockSpec((EBLK, D), lambda i: (i, 0)),
        out_shape=jax.ShapeDtypeStruct((E_PAD, D), jnp.float32),
    )(efeat_p, we, be)


def _sc_body(a_hbm, b_hbm, c_hbm, src_hbm, dst_hbm, s_out, deg_out,
             ids_v, idd_v, a_v, b_v, c_v, r_v, deg_v, acc_v, res_v,
             s_sh, deg_sh, sem_a, sem_b):
    sid = lax.axis_index("s")
    wid = sid

    def _zero_row(i, carry):
        for s in range(D // 16):
            r_v[i, pl.ds(s * 16, 16)] = jnp.zeros((16,), jnp.float32)
        return carry
    lax.fori_loop(0, CH, _zero_row, 0)

    def _zero_acc(k, carry):
        pltpu.sync_copy(r_v, s_sh.at[pl.ds(sid * ZROWS + k * CH, CH)])
        return carry
    lax.fori_loop(0, ZROWS // CH, _zero_acc, 0)
    zrem = ZROWS - (ZROWS // CH) * CH
    pltpu.sync_copy(r_v.at[pl.ds(0, zrem)],
                    s_sh.at[pl.ds(sid * ZROWS + ZROWS - zrem, zrem)])

    def _zero_deg(i, carry):
        deg_v[pl.ds(i * 16, 16)] = jnp.zeros((16,), jnp.float32)
        return carry
    lax.fori_loop(0, NP // 16, _zero_deg, 0)

    plsc.subcore_barrier()

    ones16 = jnp.ones((16,), jnp.float32)

    def _group(g, carry):
        pltpu.sync_copy(src_hbm.at[wid, g], ids_v)
        pltpu.sync_copy(dst_hbm.at[wid, g], idd_v)

        def _chunk(j, c2):
            cp_a = pltpu.async_copy(a_hbm.at[ids_v.at[j]], a_v, sem_a)
            cp_b = pltpu.async_copy(b_hbm.at[idd_v.at[j]], b_v, sem_b)
            pltpu.sync_copy(
                c_hbm.at[pl.ds(wid * PER_TILE + g * (CPG * CH) + j * CH, CH)],
                c_v)
            for k in range(CH // 16):
                idx = idd_v[j, pl.ds(k * 16, 16)]
                plsc.addupdate_scatter(deg_v, [idx], ones16)
            cp_a.wait()
            cp_b.wait()

            def _row(i, c3):
                for s in range(D // 16):
                    sl = pl.ds(s * 16, 16)
                    r_v[i, sl] = jnp.maximum(
                        a_v[i, sl] + b_v[i, sl] + c_v[i, sl], 0.0)
                return c3
            lax.fori_loop(0, CH, _row, 0)

            pltpu.sync_copy(r_v, s_sh.at[idd_v.at[j]], add=True)
            return c2
        lax.fori_loop(0, CPG, _chunk, 0)
        return carry

    lax.fori_loop(0, GROUPS, _group, 0)

    pltpu.sync_copy(deg_v, deg_sh.at[sid])
    plsc.subcore_barrier()

    pltpu.sync_copy(s_sh.at[pl.ds(sid * ROWS_PER_TILE, ROWS_PER_TILE)],
                    s_out.at[pl.ds(sid * ROWS_PER_TILE, ROWS_PER_TILE)])

    @pl.when(sid == 0)
    def _out_tail():
        pltpu.sync_copy(s_sh.at[pl.ds(NS * ROWS_PER_TILE, ROWS_REM)],
                        s_out.at[pl.ds(NS * ROWS_PER_TILE, ROWS_REM)])

    col = sid * DEG_COLS
    for t in range(NS):
        pltpu.sync_copy(deg_sh.at[t, pl.ds(col, DEG_COLS)], acc_v.at[t])

    def _red(ci, carry):
        sl = pl.ds(ci * 16, 16)
        tot = acc_v[0, sl]
        for t in range(1, NS):
            tot = tot + acc_v[t, sl]
        res_v[sl] = tot
        return carry
    lax.fori_loop(0, DEG_COLS // 16, _red, 0)

    plsc.subcore_barrier()
    pltpu.sync_copy(res_v, deg_sh.at[0, pl.ds(col, DEG_COLS)])
    plsc.subcore_barrier()

    @pl.when(sid == 0)
    def _deg_out():
        pltpu.sync_copy(deg_sh.at[0], deg_out)


def _sc_segment(a, b, c, src_r, dst_r):
    mesh = plsc.VectorSubcoreMesh(core_axis_name="c", subcore_axis_name="s",
                                  num_cores=1, num_subcores=NS)
    f = pl.kernel(
        _sc_body,
        mesh=mesh,
        out_type=[
            pltpu.HBM((N, D), jnp.float32),
            pltpu.HBM((NP,), jnp.float32),
        ],
        scratch_types=[
            pltpu.VMEM((CPG, CH), jnp.int32),
            pltpu.VMEM((CPG, CH), jnp.int32),
            pltpu.VMEM((CH, D), jnp.float32),
            pltpu.VMEM((CH, D), jnp.float32),
            pltpu.VMEM((CH, D), jnp.float32),
            pltpu.VMEM((CH, D), jnp.float32),
            pltpu.VMEM((NP,), jnp.float32),
            pltpu.VMEM((NS, DEG_COLS), jnp.float32),
            pltpu.VMEM((DEG_COLS,), jnp.float32),
            pltpu.VMEM_SHARED((SROWS, D), jnp.float32),
            pltpu.VMEM_SHARED((NS, NP), jnp.float32),
            pltpu.SemaphoreType.DMA,
            pltpu.SemaphoreType.DMA,
        ],
        compiler_params=pltpu.CompilerParams(needs_layout_passes=False),
    )
    return f(a, b, c, src_r, dst_r)


def _set2set(x_ref, w_ih, w_hh, b_ih, b_hh):
    dn = (((1,), (1,)), ((), ()))
    q_star = jnp.zeros((1, 2 * D), jnp.float32)
    h = jnp.zeros((1, D), jnp.float32)
    c = jnp.zeros((1, D), jnp.float32)
    for _ in range(3):
        gates = (lax.dot_general(q_star, w_ih, dn,
                                 preferred_element_type=jnp.float32)
                 + b_ih
                 + lax.dot_general(h, w_hh, dn,
                                   preferred_element_type=jnp.float32)
                 + b_hh)
        gi = gates[:, 0 * D:1 * D]
        gf = gates[:, 1 * D:2 * D]
        gg = gates[:, 2 * D:3 * D]
        go = gates[:, 3 * D:4 * D]
        c = jax.nn.sigmoid(gf) * c + jax.nn.sigmoid(gi) * jnp.tanh(gg)
        h = jax.nn.sigmoid(go) * jnp.tanh(c)

        def _mx(bk, m):
            blk = x_ref[pl.ds(bk * BLK, BLK), :]
            e = jnp.sum(blk * h, axis=1, keepdims=True)
            return jnp.maximum(m, jnp.max(e))
        m = lax.fori_loop(0, NBLK, _mx, jnp.float32(-1e30))

        def _acc(bk, carry):
            z, r = carry
            blk = x_ref[pl.ds(bk * BLK, BLK), :]
            e = jnp.sum(blk * h, axis=1, keepdims=True)
            p = jnp.exp(e - m)
            z = z + jnp.sum(p)
            r = r + lax.dot_general(p, blk, (((0,), (0,)), ((), ())),
                                    preferred_element_type=jnp.float32)
            return z, r
        z, r = lax.fori_loop(0, NBLK, _acc,
                             (jnp.float32(0.0), jnp.zeros((1, D), jnp.float32)))
        readout = r / z
        q_star = jnp.concatenate([h, readout], axis=1)
    return q_star


def _post_body(s_ref, deg_ref, feat_ref, wna_ref, wnb_ref, bn_ref,
               wnih_ref, wnhh_ref, bnih_ref, bnhh_ref,
               weih_ref, wehh_ref, beih_ref, behh_ref,
               w1_ref, b1_ref, w2_ref, b2_ref, w3_ref, b3_ref,
               out_ref, ne_ref, mean_ref):
    wna = wna_ref[...]
    wnb = wnb_ref[...]
    bn = bn_ref[...]

    def _blk(bk, carry):
        rows = s_ref[pl.ds(bk * BLK, BLK), :]
        deg = deg_ref[pl.ds(bk * BLK, BLK), :]
        denom = jnp.maximum(deg, 1.0)
        mean = rows / denom
        mean_ref[pl.ds(bk * BLK, BLK), :] = mean
        fb = feat_ref[pl.ds(bk * BLK, BLK), :]
        ne = jnp.dot(fb, wna, preferred_element_type=jnp.float32)
        ne = ne + jnp.dot(mean, wnb, preferred_element_type=jnp.float32)
        ne_ref[pl.ds(bk * BLK, BLK), :] = jnp.maximum(ne + bn, 0.0)
        return carry
    lax.fori_loop(0, NBLK, _blk, 0)

    node_s2s = _set2set(ne_ref, wnih_ref[...], wnhh_ref[...],
                        bnih_ref[...], bnhh_ref[...])
    edge_s2s = _set2set(mean_ref, weih_ref[...], wehh_ref[...],
                        beih_ref[...], behh_ref[...])

    out = jnp.concatenate([node_s2s, edge_s2s], axis=1)
    out = jnp.maximum(
        jnp.dot(out, w1_ref[...], preferred_element_type=jnp.float32)
        + b1_ref[...], 0.0)
    out = jnp.maximum(
        jnp.dot(out, w2_ref[...], preferred_element_type=jnp.float32)
        + b2_ref[...], 0.0)
    out_ref[...] = (
        jnp.dot(out, w3_ref[...], preferred_element_type=jnp.float32)
        + b3_ref[...])


def _post(s2, deg, feat, wna, wnb, bn, wnih, wnhh, bnih, bnhh,
          weih, wehh, beih, behh, w1, b1, w2, b2, w3, b3):
    return pl.pallas_call(
        _post_body,
        out_shape=jax.ShapeDtypeStruct((1, 16), jnp.float32),
        scratch_shapes=[
            pltpu.VMEM((N, D), jnp.float32),
            pltpu.VMEM((N, D), jnp.float32),
        ],
    )(s2, deg, feat, wna, wnb, bn, wnih, wnhh, bnih, bnhh,
      weih, wehh, beih, behh, w1, b1, w2, b2, w3, b3)


def kernel(feat, edge_index, efeat, W_edge, b_edge, W_node, b_node,
           Wn_ih, Wn_hh, bn_ih, bn_hh, We_ih, We_hh, be_ih, be_hh,
           W1, b1, W2, b2, W3, b3):
    a, b = _edge_tables_ab(feat, W_edge[:D], W_edge[D:2 * D])

    pad = E_PAD - E
    efeat_p = jnp.concatenate(
        [efeat, jnp.zeros((pad, DE), efeat.dtype)], axis=0)
    c = _edge_table_c(efeat_p, W_edge[2 * D:], b_edge.reshape(1, D))

    src_p = jnp.concatenate(
        [edge_index[0], jnp.zeros((pad,), jnp.int32)]
    ).reshape(NS, GROUPS, CPG, CH)
    dst_p = jnp.concatenate(
        [edge_index[1], jnp.full((pad,), SPAD, jnp.int32)]
    ).reshape(NS, GROUPS, CPG, CH)
    s2, deg = _sc_segment(a, b, c, src_p, dst_p)

    return _post(
        s2, deg[:N].reshape(N, 1), feat,
        W_node[:D], W_node[D:], b_node.reshape(1, D),
        Wn_ih, Wn_hh, bn_ih.reshape(1, 4 * D), bn_hh.reshape(1, 4 * D),
        We_ih, We_hh, be_ih.reshape(1, 4 * D), be_hh.reshape(1, 4 * D),
        W1, b1.reshape(1, 32), W2, b2.reshape(1, 16), W3, b3.reshape(1, 16))


# speedup vs baseline: 1.8696x; 1.1439x over previous
import jax
import jax.numpy as jnp
from jax import lax
from jax.experimental import pallas as pl
from jax.experimental.pallas import tpu as pltpu
from jax.experimental.pallas import tpu_sc as plsc

N = 10000
E = 320000
D = 128
DE = 16
NS = 16
NP = 10240
SROWS = 10112
SPAD = SROWS - 1
CH = 16
GCH = 64
GROUPS = 20
NCHUNK = GROUPS * GCH
PER_TILE = NCHUNK * CH
E_PAD = NS * PER_TILE
ROWS_PER_TILE = 624
ROWS_REM = N - NS * ROWS_PER_TILE
ZROWS = SROWS // NS
DEG_COLS = NP // NS
NBLK = 10
BLK = N // NBLK
EBLK = 1024


def _ab_body(feat_ref, ws_ref, wd_ref, a_ref, b_ref):
    f = feat_ref[...]
    a_ref[...] = jnp.dot(f, ws_ref[...], preferred_element_type=jnp.float32)
    b_ref[...] = jnp.dot(f, wd_ref[...], preferred_element_type=jnp.float32)


def _edge_tables_ab(feat, ws, wd):
    return pl.pallas_call(
        _ab_body,
        grid=(NBLK,),
        in_specs=[
            pl.BlockSpec((BLK, D), lambda i: (i, 0)),
            pl.BlockSpec((D, D), lambda i: (0, 0)),
            pl.BlockSpec((D, D), lambda i: (0, 0)),
        ],
        out_specs=[
            pl.BlockSpec((BLK, D), lambda i: (i, 0)),
            pl.BlockSpec((BLK, D), lambda i: (i, 0)),
        ],
        out_shape=[
            jax.ShapeDtypeStruct((N, D), jnp.float32),
            jax.ShapeDtypeStruct((N, D), jnp.float32),
        ],
    )(feat, ws, wd)


def _c_body(ef_ref, we_ref, be_ref, c_ref):
    c_ref[...] = (
        jnp.dot(ef_ref[...], we_ref[...], preferred_element_type=jnp.float32)
        + be_ref[...]
    )


def _edge_table_c(efeat_p, we, be):
    return pl.pallas_call(
        _c_body,
        grid=(E_PAD // EBLK,),
        in_specs=[
            pl.BlockSpec((EBLK, DE), lambda i: (i, 0)),
            pl.BlockSpec((DE, D), lambda i: (0, 0)),
            pl.BlockSpec((1, D), lambda i: (0, 0)),
        ],
        out_specs=pl.BlockSpec((EBLK, D), lambda i: (i, 0)),
        out_shape=jax.ShapeDtypeStruct((E_PAD, D), jnp.float32),
    )(efeat_p, we, be)


def _sc_body(a_hbm, b_hbm, c_hbm, src_hbm, dst_hbm, s_out, deg_out,
             idxs_v, idxd_v, a0_v, a1_v, b0_v, b1_v, c0_v, c1_v, r0_v, r1_v,
             deg_v, s_sh,
             sem_g0, sem_g1, sem_s0, sem_s1):
    sid = lax.axis_index("s")
    wid = sid
    a_v = (a0_v, a1_v)
    b_v = (b0_v, b1_v)
    c_v = (c0_v, c1_v)
    r_v = (r0_v, r1_v)
    sem_g = (sem_g0, sem_g1)
    sem_s = (sem_s0, sem_s1)

    def _zero_row(i, carry):
        for s in range(D // 16):
            r0_v[i, pl.ds(s * 16, 16)] = jnp.zeros((16,), jnp.float32)
        return carry
    lax.fori_loop(0, CH, _zero_row, 0)

    def _zero_acc(k, carry):
        pltpu.sync_copy(r0_v, s_sh.at[pl.ds(sid * ZROWS + k * CH, CH)])
        return carry
    lax.fori_loop(0, ZROWS // CH, _zero_acc, 0)
    zrem = ZROWS - (ZROWS // CH) * CH
    pltpu.sync_copy(r0_v.at[pl.ds(0, zrem)],
                    s_sh.at[pl.ds(sid * ZROWS + ZROWS - zrem, zrem)])

    def _zero_deg(i, carry):
        deg_v[pl.ds(i * 16, 16)] = jnp.zeros((16,), jnp.float32)
        return carry
    lax.fori_loop(0, NP // 16, _zero_deg, 0)

    plsc.subcore_barrier()

    ones16 = jnp.ones((16,), jnp.float32)

    def _group(g, carry):
        pltpu.sync_copy(src_hbm.at[wid, pl.ds(g * GCH, GCH)], idxs_v)
        pltpu.sync_copy(dst_hbm.at[wid, pl.ds(g * GCH, GCH)], idxd_v)
        page = 0

        def _pair(tt, c2):
            for pth in range(2):
                li_c = 2 * (tt - 1) + pth
                li_i = 2 * tt + pth

                @pl.when(tt > 0)
                def _compute():
                    @pl.when(jnp.logical_or(tt > 1, g > 0))
                    def _drain_scatter():
                        pltpu.make_async_copy(r_v[pth],
                                              s_sh.at[pl.ds(0, CH)],
                                              sem_s[pth]).wait()
                    def _drain_gathers(k, c4):
                        pltpu.make_async_copy(a_hbm.at[pl.ds(0, CH)],
                                              a_v[pth], sem_g[pth]).wait()
                        return c4
                    lax.fori_loop(0, 3, _drain_gathers, 0)

                    idx16 = idxd_v[page + li_c, pl.ds(0, 16)]
                    plsc.addupdate_scatter(deg_v, [idx16], ones16)

                    def _row(i, c3):
                        for s in range(D // 16):
                            sl = pl.ds(s * 16, 16)
                            r_v[pth][i, sl] = jnp.maximum(
                                a_v[pth][i, sl] + b_v[pth][i, sl]
                                + c_v[pth][i, sl], 0.0)
                        return c3
                    lax.fori_loop(0, CH, _row, 0)

                    pltpu.async_copy(r_v[pth],
                                     s_sh.at[idxd_v.at[page + li_c]],
                                     sem_s[pth], add=True)

                @pl.when(li_i < GCH)
                def _issue():
                    pltpu.async_copy(a_hbm.at[idxs_v.at[page + li_i]],
                                     a_v[pth], sem_g[pth])
                    pltpu.async_copy(b_hbm.at[idxd_v.at[page + li_i]],
                                     b_v[pth], sem_g[pth])
                    pltpu.async_copy(
                        c_hbm.at[pl.ds(
                            wid * PER_TILE + (g * GCH + li_i) * CH, CH)],
                        c_v[pth], sem_g[pth])
            return c2

        lax.fori_loop(0, GCH // 2 + 1, _pair, 0)
        return carry

    lax.fori_loop(0, GROUPS, _group, 0)

    for pth in range(2):
        pltpu.make_async_copy(r_v[pth], s_sh.at[pl.ds(0, CH)],
                              sem_s[pth]).wait()

    pltpu.sync_copy(deg_v, deg_out.at[sid])

    plsc.subcore_barrier()

    pltpu.sync_copy(s_sh.at[pl.ds(sid * ROWS_PER_TILE, ROWS_PER_TILE)],
                    s_out.at[pl.ds(sid * ROWS_PER_TILE, ROWS_PER_TILE)])

    @pl.when(sid == 0)
    def _out_tail():
        pltpu.sync_copy(s_sh.at[pl.ds(NS * ROWS_PER_TILE, ROWS_REM)],
                        s_out.at[pl.ds(NS * ROWS_PER_TILE, ROWS_REM)])


def _sc_segment(a, b, c, src_r, dst_r):
    mesh = plsc.VectorSubcoreMesh(core_axis_name="c", subcore_axis_name="s",
                                  num_cores=1, num_subcores=NS)
    f = pl.kernel(
        _sc_body,
        mesh=mesh,
        out_type=[
            pltpu.HBM((N, D), jnp.float32),
            pltpu.HBM((NS, NP), jnp.float32),
        ],
        scratch_types=[
            pltpu.VMEM((GCH, CH), jnp.int32),
            pltpu.VMEM((GCH, CH), jnp.int32),
            pltpu.VMEM((CH, D), jnp.float32),
            pltpu.VMEM((CH, D), jnp.float32),
            pltpu.VMEM((CH, D), jnp.float32),
            pltpu.VMEM((CH, D), jnp.float32),
            pltpu.VMEM((CH, D), jnp.float32),
            pltpu.VMEM((CH, D), jnp.float32),
            pltpu.VMEM((CH, D), jnp.float32),
            pltpu.VMEM((CH, D), jnp.float32),
            pltpu.VMEM((NP,), jnp.float32),
            pltpu.VMEM_SHARED((SROWS, D), jnp.float32),
            pltpu.SemaphoreType.DMA,
            pltpu.SemaphoreType.DMA,
            pltpu.SemaphoreType.DMA,
            pltpu.SemaphoreType.DMA,
        ],
        compiler_params=pltpu.CompilerParams(needs_layout_passes=False),
    )
    return f(a, b, c, src_r, dst_r)


def _degt_body(degp_ref, out_ref):
    tot = lax.dot_general(jnp.ones((1, NS), jnp.float32), degp_ref[...],
                          (((1,), (0,)), ((), ())),
                          preferred_element_type=jnp.float32)
    ri = lax.broadcasted_iota(jnp.int32, (128, 128), 0)
    ci = lax.broadcasted_iota(jnp.int32, (128, 128), 1)
    ident = (ri == ci).astype(jnp.float32)
    out_ref[...] = lax.dot_general(ident, tot, (((1,), (1,)), ((), ())),
                                   preferred_element_type=jnp.float32)


def _deg_transpose(deg_p):
    return pl.pallas_call(
        _degt_body,
        grid=(NP // 128,),
        in_specs=[pl.BlockSpec((NS, 128), lambda i: (0, i))],
        out_specs=pl.BlockSpec((128, 1), lambda i: (i, 0)),
        out_shape=jax.ShapeDtypeStruct((NP, 1), jnp.float32),
    )(deg_p)


def _set2set(x_ref, w_ih, w_hh, b_ih, b_hh):
    dn = (((1,), (1,)), ((), ()))
    q_star = jnp.zeros((1, 2 * D), jnp.float32)
    h = jnp.zeros((1, D), jnp.float32)
    c = jnp.zeros((1, D), jnp.float32)
    for _ in range(3):
        gates = (lax.dot_general(q_star, w_ih, dn,
                                 preferred_element_type=jnp.float32)
                 + b_ih
                 + lax.dot_general(h, w_hh, dn,
                                   preferred_element_type=jnp.float32)
                 + b_hh)
        gi = gates[:, 0 * D:1 * D]
        gf = gates[:, 1 * D:2 * D]
        gg = gates[:, 2 * D:3 * D]
        go = gates[:, 3 * D:4 * D]
        c = jax.nn.sigmoid(gf) * c + jax.nn.sigmoid(gi) * jnp.tanh(gg)
        h = jax.nn.sigmoid(go) * jnp.tanh(c)

        def _mx(bk, m):
            blk = x_ref[pl.ds(bk * BLK, BLK), :]
            e = jnp.sum(blk * h, axis=1, keepdims=True)
            return jnp.maximum(m, jnp.max(e))
        m = lax.fori_loop(0, NBLK, _mx, jnp.float32(-1e30))

        def _acc(bk, carry):
            z, r = carry
            blk = x_ref[pl.ds(bk * BLK, BLK), :]
            e = jnp.sum(blk * h, axis=1, keepdims=True)
            p = jnp.exp(e - m)
            z = z + jnp.sum(p)
            r = r + lax.dot_general(p, blk, (((0,), (0,)), ((), ())),
                                    preferred_element_type=jnp.float32)
            return z, r
        z, r = lax.fori_loop(0, NBLK, _acc,
                             (jnp.float32(0.0), jnp.zeros((1, D), jnp.float32)))
        readout = r / z
        q_star = jnp.concatenate([h, readout], axis=1)
    return q_star


def _post_body(s_ref, deg_ref, feat_ref, wna_ref, wnb_ref, bn_ref,
               wnih_ref, wnhh_ref, bnih_ref, bnhh_ref,
               weih_ref, wehh_ref, beih_ref, behh_ref,
               w1_ref, b1_ref, w2_ref, b2_ref, w3_ref, b3_ref,
               out_ref, ne_ref, mean_ref):
    wna = wna_ref[...]
    wnb = wnb_ref[...]
    bn = bn_ref[...]

    def _blk(bk, carry):
        rows = s_ref[pl.ds(bk * BLK, BLK), :]
        deg = deg_ref[pl.ds(bk * BLK, BLK), :]
        denom = jnp.maximum(deg, 1.0)
        mean = rows / denom
        mean_ref[pl.ds(bk * BLK, BLK), :] = mean
        fb = feat_ref[pl.ds(bk * BLK, BLK), :]
        ne = jnp.dot(fb, wna, preferred_element_type=jnp.float32)
        ne = ne + jnp.dot(mean, wnb, preferred_element_type=jnp.float32)
        ne_ref[pl.ds(bk * BLK, BLK), :] = jnp.maximum(ne + bn, 0.0)
        return carry
    lax.fori_loop(0, NBLK, _blk, 0)

    node_s2s = _set2set(ne_ref, wnih_ref[...], wnhh_ref[...],
                        bnih_ref[...], bnhh_ref[...])
    edge_s2s = _set2set(mean_ref, weih_ref[...], wehh_ref[...],
                        beih_ref[...], behh_ref[...])

    out = jnp.concatenate([node_s2s, edge_s2s], axis=1)
    out = jnp.maximum(
        jnp.dot(out, w1_ref[...], preferred_element_type=jnp.float32)
        + b1_ref[...], 0.0)
    out = jnp.maximum(
        jnp.dot(out, w2_ref[...], preferred_element_type=jnp.float32)
        + b2_ref[...], 0.0)
    out_ref[...] = (
        jnp.dot(out, w3_ref[...], preferred_element_type=jnp.float32)
        + b3_ref[...])


def _post(s2, deg, feat, wna, wnb, bn, wnih, wnhh, bnih, bnhh,
          weih, wehh, beih, behh, w1, b1, w2, b2, w3, b3):
    return pl.pallas_call(
        _post_body,
        out_shape=jax.ShapeDtypeStruct((1, 16), jnp.float32),
        scratch_shapes=[
            pltpu.VMEM((N, D), jnp.float32),
            pltpu.VMEM((N, D), jnp.float32),
        ],
    )(s2, deg, feat, wna, wnb, bn, wnih, wnhh, bnih, bnhh,
      weih, wehh, beih, behh, w1, b1, w2, b2, w3, b3)


def kernel(feat, edge_index, efeat, W_edge, b_edge, W_node, b_node,
           Wn_ih, Wn_hh, bn_ih, bn_hh, We_ih, We_hh, be_ih, be_hh,
           W1, b1, W2, b2, W3, b3):
    a, b = _edge_tables_ab(feat, W_edge[:D], W_edge[D:2 * D])

    pad = E_PAD - E
    efeat_p = jnp.concatenate(
        [efeat, jnp.zeros((pad, DE), efeat.dtype)], axis=0)
    c = _edge_table_c(efeat_p, W_edge[2 * D:], b_edge.reshape(1, D))

    src_p = jnp.concatenate(
        [edge_index[0], jnp.zeros((pad,), jnp.int32)]
    ).reshape(NS, NCHUNK, CH)
    dst_p = jnp.concatenate(
        [edge_index[1], jnp.full((pad,), SPAD, jnp.int32)]
    ).reshape(NS, NCHUNK, CH)
    s2, deg = _sc_segment(a, b, c, src_p, dst_p)

    deg_col = _deg_transpose(deg)[:N]
    return _post(
        s2, deg_col, feat,
        W_node[:D], W_node[D:], b_node.reshape(1, D),
        Wn_ih, Wn_hh, bn_ih.reshape(1, 4 * D), bn_hh.reshape(1, 4 * D),
        We_ih, We_hh, be_ih.reshape(1, 4 * D), be_hh.reshape(1, 4 * D),
        W1, b1.reshape(1, 32), W2, b2.reshape(1, 16), W3, b3.reshape(1, 16))


# speedup vs baseline: 1.9064x; 1.0197x over previous
import jax
import jax.numpy as jnp
from jax import lax
from jax.experimental import pallas as pl
from jax.experimental.pallas import tpu as pltpu
from jax.experimental.pallas import tpu_sc as plsc

N = 10000
E = 320000
D = 128
DE = 16
NS = 16
NP = 10240
SROWS = 10112
SPAD = SROWS - 1
CH = 16
GCH = 64
GROUPS = 10
NCHUNK = GROUPS * GCH
PER_TILE = NCHUNK * CH
NSPLIT = 2
E_CALL = NS * PER_TILE
E_PAD = NSPLIT * E_CALL
ROWS_PER_TILE = 624
ROWS_REM = N - NS * ROWS_PER_TILE
ZROWS = SROWS // NS
DEG_COLS = NP // NS
NBLK = 10
BLK = N // NBLK
EBLK = 1024


def _ab_body(feat_ref, ws_ref, wd_ref, a_ref, b_ref):
    f = feat_ref[...]
    a_ref[...] = jnp.dot(f, ws_ref[...], preferred_element_type=jnp.float32)
    b_ref[...] = jnp.dot(f, wd_ref[...], preferred_element_type=jnp.float32)


def _edge_tables_ab(feat, ws, wd):
    return pl.pallas_call(
        _ab_body,
        grid=(NBLK,),
        in_specs=[
            pl.BlockSpec((BLK, D), lambda i: (i, 0)),
            pl.BlockSpec((D, D), lambda i: (0, 0)),
            pl.BlockSpec((D, D), lambda i: (0, 0)),
        ],
        out_specs=[
            pl.BlockSpec((BLK, D), lambda i: (i, 0)),
            pl.BlockSpec((BLK, D), lambda i: (i, 0)),
        ],
        out_shape=[
            jax.ShapeDtypeStruct((N, D), jnp.float32),
            jax.ShapeDtypeStruct((N, D), jnp.float32),
        ],
    )(feat, ws, wd)


def _c_body(ef_ref, we_ref, be_ref, c_ref):
    c_ref[...] = (
        jnp.dot(ef_ref[...], we_ref[...], preferred_element_type=jnp.float32)
        + be_ref[...]
    )


def _edge_table_c(efeat_p, we, be):
    return pl.pallas_call(
        _c_body,
        grid=(E_PAD // EBLK,),
        in_specs=[
            pl.BlockSpec((EBLK, DE), lambda i: (i, 0)),
            pl.BlockSpec((DE, D), lambda i: (0, 0)),
            pl.BlockSpec((1, D), lambda i: (0, 0)),
        ],
        out_specs=pl.BlockSpec((EBLK, D), lambda i: (i, 0)),
        out_shape=jax.ShapeDtypeStruct((E_PAD, D), jnp.float32),
    )(efeat_p, we, be)


def _make_sc_body(base):
    def _sc_body(a_hbm, b_hbm, c_hbm, src_hbm, dst_hbm, s_out, deg_out,
                 idxs_v, idxd_v, a0_v, a1_v, b0_v, b1_v, c0_v, c1_v, r0_v, r1_v,
                 deg_v, s_sh,
                 sem_g0, sem_g1, sem_s0, sem_s1):
        sid = lax.axis_index("s")
        wid = sid
        a_v = (a0_v, a1_v)
        b_v = (b0_v, b1_v)
        c_v = (c0_v, c1_v)
        r_v = (r0_v, r1_v)
        sem_g = (sem_g0, sem_g1)
        sem_s = (sem_s0, sem_s1)

        def _zero_row(i, carry):
            for s in range(D // 16):
                r0_v[i, pl.ds(s * 16, 16)] = jnp.zeros((16,), jnp.float32)
            return carry
        lax.fori_loop(0, CH, _zero_row, 0)

        def _zero_acc(k, carry):
            pltpu.sync_copy(r0_v, s_sh.at[pl.ds(sid * ZROWS + k * CH, CH)])
            return carry
        lax.fori_loop(0, ZROWS // CH, _zero_acc, 0)
        zrem = ZROWS - (ZROWS // CH) * CH
        pltpu.sync_copy(r0_v.at[pl.ds(0, zrem)],
                        s_sh.at[pl.ds(sid * ZROWS + ZROWS - zrem, zrem)])

        def _zero_deg(i, carry):
            deg_v[pl.ds(i * 16, 16)] = jnp.zeros((16,), jnp.float32)
            return carry
        lax.fori_loop(0, NP // 16, _zero_deg, 0)

        plsc.subcore_barrier()

        ones16 = jnp.ones((16,), jnp.float32)

        def _group(g, carry):
            pltpu.sync_copy(src_hbm.at[wid, pl.ds(g * GCH, GCH)], idxs_v)
            pltpu.sync_copy(dst_hbm.at[wid, pl.ds(g * GCH, GCH)], idxd_v)
            page = 0

            def _pair(tt, c2):
                for pth in range(2):
                    li_c = 2 * (tt - 1) + pth
                    li_i = 2 * tt + pth

                    @pl.when(tt > 0)
                    def _compute():
                        @pl.when(jnp.logical_or(tt > 1, g > 0))
                        def _drain_scatter():
                            pltpu.make_async_copy(r_v[pth],
                                                  s_sh.at[pl.ds(0, CH)],
                                                  sem_s[pth]).wait()
                        def _drain_gathers(k, c4):
                            pltpu.make_async_copy(a_hbm.at[pl.ds(0, CH)],
                                                  a_v[pth], sem_g[pth]).wait()
                            return c4
                        lax.fori_loop(0, 3, _drain_gathers, 0)

                        idx16 = idxd_v[page + li_c, pl.ds(0, 16)]
                        plsc.addupdate_scatter(deg_v, [idx16], ones16)

                        def _row(i, c3):
                            for s in range(D // 16):
                                sl = pl.ds(s * 16, 16)
                                r_v[pth][i, sl] = jnp.maximum(
                                    a_v[pth][i, sl] + b_v[pth][i, sl]
                                    + c_v[pth][i, sl], 0.0)
                            return c3
                        lax.fori_loop(0, CH, _row, 0)

                        pltpu.async_copy(r_v[pth],
                                         s_sh.at[idxd_v.at[page + li_c]],
                                         sem_s[pth], add=True)

                    @pl.when(li_i < GCH)
                    def _issue():
                        pltpu.async_copy(a_hbm.at[idxs_v.at[page + li_i]],
                                         a_v[pth], sem_g[pth])
                        pltpu.async_copy(b_hbm.at[idxd_v.at[page + li_i]],
                                         b_v[pth], sem_g[pth])
                        pltpu.async_copy(
                            c_hbm.at[pl.ds(
                                base + wid * PER_TILE
                                + (g * GCH + li_i) * CH, CH)],
                            c_v[pth], sem_g[pth])
                return c2

            lax.fori_loop(0, GCH // 2 + 1, _pair, 0)
            return carry

        lax.fori_loop(0, GROUPS, _group, 0)

        for pth in range(2):
            pltpu.make_async_copy(r_v[pth], s_sh.at[pl.ds(0, CH)],
                                  sem_s[pth]).wait()

        pltpu.sync_copy(deg_v, deg_out.at[sid])

        plsc.subcore_barrier()

        pltpu.sync_copy(s_sh.at[pl.ds(sid * ROWS_PER_TILE, ROWS_PER_TILE)],
                        s_out.at[pl.ds(sid * ROWS_PER_TILE, ROWS_PER_TILE)])

        @pl.when(sid == 0)
        def _out_tail():
            pltpu.sync_copy(s_sh.at[pl.ds(NS * ROWS_PER_TILE, ROWS_REM)],
                            s_out.at[pl.ds(NS * ROWS_PER_TILE, ROWS_REM)])


    return _sc_body


def _sc_segment(a, b, c, src_r, dst_r, base):
    mesh = plsc.VectorSubcoreMesh(core_axis_name="c", subcore_axis_name="s",
                                  num_cores=1, num_subcores=NS)
    f = pl.kernel(
        _make_sc_body(base),
        mesh=mesh,
        out_type=[
            pltpu.HBM((N, D), jnp.float32),
            pltpu.HBM((NS, NP), jnp.float32),
        ],
        scratch_types=[
            pltpu.VMEM((GCH, CH), jnp.int32),
            pltpu.VMEM((GCH, CH), jnp.int32),
            pltpu.VMEM((CH, D), jnp.float32),
            pltpu.VMEM((CH, D), jnp.float32),
            pltpu.VMEM((CH, D), jnp.float32),
            pltpu.VMEM((CH, D), jnp.float32),
            pltpu.VMEM((CH, D), jnp.float32),
            pltpu.VMEM((CH, D), jnp.float32),
            pltpu.VMEM((CH, D), jnp.float32),
            pltpu.VMEM((CH, D), jnp.float32),
            pltpu.VMEM((NP,), jnp.float32),
            pltpu.VMEM_SHARED((SROWS, D), jnp.float32),
            pltpu.SemaphoreType.DMA,
            pltpu.SemaphoreType.DMA,
            pltpu.SemaphoreType.DMA,
            pltpu.SemaphoreType.DMA,
        ],
        compiler_params=pltpu.CompilerParams(needs_layout_passes=False),
    )
    return f(a, b, c, src_r, dst_r)


def _degt_body(degp_ref, out_ref):
    tot = lax.dot_general(jnp.ones((1, 2 * NS), jnp.float32), degp_ref[...],
                          (((1,), (0,)), ((), ())),
                          preferred_element_type=jnp.float32)
    ri = lax.broadcasted_iota(jnp.int32, (128, 128), 0)
    ci = lax.broadcasted_iota(jnp.int32, (128, 128), 1)
    ident = (ri == ci).astype(jnp.float32)
    out_ref[...] = lax.dot_general(ident, tot, (((1,), (1,)), ((), ())),
                                   preferred_element_type=jnp.float32)


def _deg_transpose(deg_p):
    return pl.pallas_call(
        _degt_body,
        grid=(NP // 128,),
        in_specs=[pl.BlockSpec((2 * NS, 128), lambda i: (0, i))],
        out_specs=pl.BlockSpec((128, 1), lambda i: (i, 0)),
        out_shape=jax.ShapeDtypeStruct((NP, 1), jnp.float32),
    )(deg_p)


def _set2set(x_ref, w_ih, w_hh, b_ih, b_hh):
    dn = (((1,), (1,)), ((), ()))
    q_star = jnp.zeros((1, 2 * D), jnp.float32)
    h = jnp.zeros((1, D), jnp.float32)
    c = jnp.zeros((1, D), jnp.float32)
    for _ in range(3):
        gates = (lax.dot_general(q_star, w_ih, dn,
                                 preferred_element_type=jnp.float32)
                 + b_ih
                 + lax.dot_general(h, w_hh, dn,
                                   preferred_element_type=jnp.float32)
                 + b_hh)
        gi = gates[:, 0 * D:1 * D]
        gf = gates[:, 1 * D:2 * D]
        gg = gates[:, 2 * D:3 * D]
        go = gates[:, 3 * D:4 * D]
        c = jax.nn.sigmoid(gf) * c + jax.nn.sigmoid(gi) * jnp.tanh(gg)
        h = jax.nn.sigmoid(go) * jnp.tanh(c)

        def _mx(bk, m):
            blk = x_ref[pl.ds(bk * BLK, BLK), :]
            e = jnp.sum(blk * h, axis=1, keepdims=True)
            return jnp.maximum(m, jnp.max(e))
        m = lax.fori_loop(0, NBLK, _mx, jnp.float32(-1e30))

        def _acc(bk, carry):
            z, r = carry
            blk = x_ref[pl.ds(bk * BLK, BLK), :]
            e = jnp.sum(blk * h, axis=1, keepdims=True)
            p = jnp.exp(e - m)
            z = z + jnp.sum(p)
            r = r + lax.dot_general(p, blk, (((0,), (0,)), ((), ())),
                                    preferred_element_type=jnp.float32)
            return z, r
        z, r = lax.fori_loop(0, NBLK, _acc,
                             (jnp.float32(0.0), jnp.zeros((1, D), jnp.float32)))
        readout = r / z
        q_star = jnp.concatenate([h, readout], axis=1)
    return q_star


def _post_body(sa_ref, sb_ref, deg_ref, feat_ref, wna_ref, wnb_ref, bn_ref,
               wnih_ref, wnhh_ref, bnih_ref, bnhh_ref,
               weih_ref, wehh_ref, beih_ref, behh_ref,
               w1_ref, b1_ref, w2_ref, b2_ref, w3_ref, b3_ref,
               out_ref, ne_ref, mean_ref):
    wna = wna_ref[...]
    wnb = wnb_ref[...]
    bn = bn_ref[...]

    def _blk(bk, carry):
        rows = (sa_ref[pl.ds(bk * BLK, BLK), :]
                + sb_ref[pl.ds(bk * BLK, BLK), :])
        deg = deg_ref[pl.ds(bk * BLK, BLK), :]
        denom = jnp.maximum(deg, 1.0)
        mean = rows / denom
        mean_ref[pl.ds(bk * BLK, BLK), :] = mean
        fb = feat_ref[pl.ds(bk * BLK, BLK), :]
        ne = jnp.dot(fb, wna, preferred_element_type=jnp.float32)
        ne = ne + jnp.dot(mean, wnb, preferred_element_type=jnp.float32)
        ne_ref[pl.ds(bk * BLK, BLK), :] = jnp.maximum(ne + bn, 0.0)
        return carry
    lax.fori_loop(0, NBLK, _blk, 0)

    node_s2s = _set2set(ne_ref, wnih_ref[...], wnhh_ref[...],
                        bnih_ref[...], bnhh_ref[...])
    edge_s2s = _set2set(mean_ref, weih_ref[...], wehh_ref[...],
                        beih_ref[...], behh_ref[...])

    out = jnp.concatenate([node_s2s, edge_s2s], axis=1)
    out = jnp.maximum(
        jnp.dot(out, w1_ref[...], preferred_element_type=jnp.float32)
        + b1_ref[...], 0.0)
    out = jnp.maximum(
        jnp.dot(out, w2_ref[...], preferred_element_type=jnp.float32)
        + b2_ref[...], 0.0)
    out_ref[...] = (
        jnp.dot(out, w3_ref[...], preferred_element_type=jnp.float32)
        + b3_ref[...])


def _post(sa, sb, deg, feat, wna, wnb, bn, wnih, wnhh, bnih, bnhh,
          weih, wehh, beih, behh, w1, b1, w2, b2, w3, b3):
    return pl.pallas_call(
        _post_body,
        out_shape=jax.ShapeDtypeStruct((1, 16), jnp.float32),
        scratch_shapes=[
            pltpu.VMEM((N, D), jnp.float32),
            pltpu.VMEM((N, D), jnp.float32),
        ],
    )(sa, sb, deg, feat, wna, wnb, bn, wnih, wnhh, bnih, bnhh,
      weih, wehh, beih, behh, w1, b1, w2, b2, w3, b3)


def kernel(feat, edge_index, efeat, W_edge, b_edge, W_node, b_node,
           Wn_ih, Wn_hh, bn_ih, bn_hh, We_ih, We_hh, be_ih, be_hh,
           W1, b1, W2, b2, W3, b3):
    a, b = _edge_tables_ab(feat, W_edge[:D], W_edge[D:2 * D])

    pad = E_PAD - E
    efeat_p = jnp.concatenate(
        [efeat, jnp.zeros((pad, DE), efeat.dtype)], axis=0)
    c = _edge_table_c(efeat_p, W_edge[2 * D:], b_edge.reshape(1, D))

    src_p = jnp.concatenate([edge_index[0], jnp.zeros((pad,), jnp.int32)])
    dst_p = jnp.concatenate([edge_index[1], jnp.full((pad,), SPAD, jnp.int32)])
    sa, dega = _sc_segment(
        a, b, c, src_p[:E_CALL].reshape(NS, NCHUNK, CH),
        dst_p[:E_CALL].reshape(NS, NCHUNK, CH), 0)
    sb, degb = _sc_segment(
        a, b, c, src_p[E_CALL:].reshape(NS, NCHUNK, CH),
        dst_p[E_CALL:].reshape(NS, NCHUNK, CH), E_CALL)

    deg_col = _deg_transpose(jnp.concatenate([dega, degb], axis=0))[:N]
    return _post(
        sa, sb, deg_col, feat,
        W_node[:D], W_node[D:], b_node.reshape(1, D),
        Wn_ih, Wn_hh, bn_ih.reshape(1, 4 * D), bn_hh.reshape(1, 4 * D),
        We_ih, We_hh, be_ih.reshape(1, 4 * D), be_hh.reshape(1, 4 * D),
        W1, b1.reshape(1, 32), W2, b2.reshape(1, 16), W3, b3.reshape(1, 16))


# speedup vs baseline: 2.0968x; 1.0999x over previous
import jax
import jax.numpy as jnp
from jax import lax
from jax.experimental import pallas as pl
from jax.experimental.pallas import tpu as pltpu
from jax.experimental.pallas import tpu_sc as plsc

N = 10000
E = 320000
D = 128
DE = 16
NS = 16
NP = 10240
SROWS = 10112
SPAD = SROWS - 1
CH = 16
GCH = 64
GROUPS = 10
NCHUNK = GROUPS * GCH
PER_TILE = NCHUNK * CH
NSPLIT = 2
E_CALL = NS * PER_TILE
E_PAD = NSPLIT * E_CALL
ROWS_PER_TILE = 624
ROWS_REM = N - NS * ROWS_PER_TILE
ZROWS = SROWS // NS
DEG_COLS = NP // NS
NBLK = 10
BLK = N // NBLK
EBLK = 4096


def _ab_body(feat_ref, ws_ref, wd_ref, a_ref, b_ref):
    f = feat_ref[...]
    a_ref[...] = jnp.dot(f, ws_ref[...], preferred_element_type=jnp.float32)
    b_ref[...] = jnp.dot(f, wd_ref[...], preferred_element_type=jnp.float32)


def _edge_tables_ab(feat, ws, wd):
    return pl.pallas_call(
        _ab_body,
        grid=(NBLK,),
        in_specs=[
            pl.BlockSpec((BLK, D), lambda i: (i, 0)),
            pl.BlockSpec((D, D), lambda i: (0, 0)),
            pl.BlockSpec((D, D), lambda i: (0, 0)),
        ],
        out_specs=[
            pl.BlockSpec((BLK, D), lambda i: (i, 0)),
            pl.BlockSpec((BLK, D), lambda i: (i, 0)),
        ],
        out_shape=[
            jax.ShapeDtypeStruct((N, D), jnp.float32),
            jax.ShapeDtypeStruct((N, D), jnp.float32),
        ],
    )(feat, ws, wd)


def _c_body(ef_ref, we_ref, be_ref, c_ref):
    c_ref[...] = (
        jnp.dot(ef_ref[...], we_ref[...], preferred_element_type=jnp.float32)
        + be_ref[...]
    )


def _edge_table_c(efeat_p, we, be):
    return pl.pallas_call(
        _c_body,
        grid=(E_PAD // EBLK,),
        in_specs=[
            pl.BlockSpec((EBLK, DE), lambda i: (i, 0)),
            pl.BlockSpec((DE, D), lambda i: (0, 0)),
            pl.BlockSpec((1, D), lambda i: (0, 0)),
        ],
        out_specs=pl.BlockSpec((EBLK, D), lambda i: (i, 0)),
        out_shape=jax.ShapeDtypeStruct((E_PAD, D), jnp.float32),
    )(efeat_p, we, be)


def _make_sc_body(base):
    def _sc_body(a_hbm, b_hbm, c_hbm, src_hbm, dst_hbm, s_out, deg_out,
                 idxs_v, idxd_v, a0_v, a1_v, b0_v, b1_v, c0_v, c1_v, r0_v, r1_v,
                 deg_v, s_sh,
                 sem_g0, sem_g1, sem_s0, sem_s1):
        sid = lax.axis_index("s")
        wid = sid
        a_v = (a0_v, a1_v)
        b_v = (b0_v, b1_v)
        c_v = (c0_v, c1_v)
        r_v = (r0_v, r1_v)
        sem_g = (sem_g0, sem_g1)
        sem_s = (sem_s0, sem_s1)

        def _zero_row(i, carry):
            for s in range(D // 16):
                r0_v[i, pl.ds(s * 16, 16)] = jnp.zeros((16,), jnp.float32)
            return carry
        lax.fori_loop(0, CH, _zero_row, 0)

        def _zero_acc(k, carry):
            pltpu.sync_copy(r0_v, s_sh.at[pl.ds(sid * ZROWS + k * CH, CH)])
            return carry
        lax.fori_loop(0, ZROWS // CH, _zero_acc, 0)
        zrem = ZROWS - (ZROWS // CH) * CH
        pltpu.sync_copy(r0_v.at[pl.ds(0, zrem)],
                        s_sh.at[pl.ds(sid * ZROWS + ZROWS - zrem, zrem)])

        def _zero_deg(i, carry):
            deg_v[pl.ds(i * 16, 16)] = jnp.zeros((16,), jnp.float32)
            return carry
        lax.fori_loop(0, NP // 16, _zero_deg, 0)

        plsc.subcore_barrier()

        ones16 = jnp.ones((16,), jnp.float32)

        def _group(g, carry):
            pltpu.sync_copy(src_hbm.at[wid, pl.ds(g * GCH, GCH)], idxs_v)
            pltpu.sync_copy(dst_hbm.at[wid, pl.ds(g * GCH, GCH)], idxd_v)
            page = 0

            def _pair(tt, c2):
                for pth in range(2):
                    li_c = 2 * (tt - 1) + pth
                    li_i = 2 * tt + pth

                    @pl.when(tt > 0)
                    def _compute():
                        @pl.when(jnp.logical_or(tt > 1, g > 0))
                        def _drain_scatter():
                            pltpu.make_async_copy(r_v[pth],
                                                  s_sh.at[pl.ds(0, CH)],
                                                  sem_s[pth]).wait()
                        def _drain_gathers(k, c4):
                            pltpu.make_async_copy(a_hbm.at[pl.ds(0, CH)],
                                                  a_v[pth], sem_g[pth]).wait()
                            return c4
                        lax.fori_loop(0, 3, _drain_gathers, 0)

                        idx16 = idxd_v[page + li_c, pl.ds(0, 16)]
                        plsc.addupdate_scatter(deg_v, [idx16], ones16)

                        def _row(i, c3):
                            for s in range(D // 16):
                                sl = pl.ds(s * 16, 16)
                                r_v[pth][i, sl] = jnp.maximum(
                                    a_v[pth][i, sl] + b_v[pth][i, sl]
                                    + c_v[pth][i, sl], 0.0)
                            return c3
                        lax.fori_loop(0, CH, _row, 0)

                        pltpu.async_copy(r_v[pth],
                                         s_sh.at[idxd_v.at[page + li_c]],
                                         sem_s[pth], add=True)

                    @pl.when(li_i < GCH)
                    def _issue():
                        pltpu.async_copy(a_hbm.at[idxs_v.at[page + li_i]],
                                         a_v[pth], sem_g[pth])
                        pltpu.async_copy(b_hbm.at[idxd_v.at[page + li_i]],
                                         b_v[pth], sem_g[pth])
                        pltpu.async_copy(
                            c_hbm.at[pl.ds(
                                base + wid * PER_TILE
                                + (g * GCH + li_i) * CH, CH)],
                            c_v[pth], sem_g[pth])
                return c2

            lax.fori_loop(0, GCH // 2 + 1, _pair, 0)
            return carry

        lax.fori_loop(0, GROUPS, _group, 0)

        for pth in range(2):
            pltpu.make_async_copy(r_v[pth], s_sh.at[pl.ds(0, CH)],
                                  sem_s[pth]).wait()

        pltpu.sync_copy(deg_v, deg_out.at[sid])

        plsc.subcore_barrier()

        pltpu.sync_copy(s_sh.at[pl.ds(sid * ROWS_PER_TILE, ROWS_PER_TILE)],
                        s_out.at[pl.ds(sid * ROWS_PER_TILE, ROWS_PER_TILE)])

        @pl.when(sid == 0)
        def _out_tail():
            pltpu.sync_copy(s_sh.at[pl.ds(NS * ROWS_PER_TILE, ROWS_REM)],
                            s_out.at[pl.ds(NS * ROWS_PER_TILE, ROWS_REM)])


    return _sc_body


def _sc_segment(a, b, c, src_r, dst_r, base):
    mesh = plsc.VectorSubcoreMesh(core_axis_name="c", subcore_axis_name="s",
                                  num_cores=1, num_subcores=NS)
    f = pl.kernel(
        _make_sc_body(base),
        mesh=mesh,
        out_type=[
            pltpu.HBM((N, D), jnp.float32),
            pltpu.HBM((NS, NP), jnp.float32),
        ],
        scratch_types=[
            pltpu.VMEM((GCH, CH), jnp.int32),
            pltpu.VMEM((GCH, CH), jnp.int32),
            pltpu.VMEM((CH, D), jnp.float32),
            pltpu.VMEM((CH, D), jnp.float32),
            pltpu.VMEM((CH, D), jnp.float32),
            pltpu.VMEM((CH, D), jnp.float32),
            pltpu.VMEM((CH, D), jnp.float32),
            pltpu.VMEM((CH, D), jnp.float32),
            pltpu.VMEM((CH, D), jnp.float32),
            pltpu.VMEM((CH, D), jnp.float32),
            pltpu.VMEM((NP,), jnp.float32),
            pltpu.VMEM_SHARED((SROWS, D), jnp.float32),
            pltpu.SemaphoreType.DMA,
            pltpu.SemaphoreType.DMA,
            pltpu.SemaphoreType.DMA,
            pltpu.SemaphoreType.DMA,
        ],
        compiler_params=pltpu.CompilerParams(needs_layout_passes=False),
    )
    return f(a, b, c, src_r, dst_r)


def _degt_body(degp_ref, out_ref):
    tot = lax.dot_general(jnp.ones((1, 2 * NS), jnp.float32), degp_ref[...],
                          (((1,), (0,)), ((), ())),
                          preferred_element_type=jnp.float32)
    ri = lax.broadcasted_iota(jnp.int32, (128, 128), 0)
    ci = lax.broadcasted_iota(jnp.int32, (128, 128), 1)
    ident = (ri == ci).astype(jnp.float32)
    out_ref[...] = lax.dot_general(ident, tot, (((1,), (1,)), ((), ())),
                                   preferred_element_type=jnp.float32)


def _deg_transpose(deg_p):
    return pl.pallas_call(
        _degt_body,
        grid=(NP // 128,),
        in_specs=[pl.BlockSpec((2 * NS, 128), lambda i: (0, i))],
        out_specs=pl.BlockSpec((128, 1), lambda i: (i, 0)),
        out_shape=jax.ShapeDtypeStruct((NP, 1), jnp.float32),
    )(deg_p)


def _set2set(x_ref, w_ih, w_hh, b_ih, b_hh):
    dn = (((1,), (1,)), ((), ()))
    q_star = jnp.zeros((1, 2 * D), jnp.float32)
    h = jnp.zeros((1, D), jnp.float32)
    c = jnp.zeros((1, D), jnp.float32)
    for _ in range(3):
        gates = (lax.dot_general(q_star, w_ih, dn,
                                 preferred_element_type=jnp.float32)
                 + b_ih
                 + lax.dot_general(h, w_hh, dn,
                                   preferred_element_type=jnp.float32)
                 + b_hh)
        gi = gates[:, 0 * D:1 * D]
        gf = gates[:, 1 * D:2 * D]
        gg = gates[:, 2 * D:3 * D]
        go = gates[:, 3 * D:4 * D]
        c = jax.nn.sigmoid(gf) * c + jax.nn.sigmoid(gi) * jnp.tanh(gg)
        h = jax.nn.sigmoid(go) * jnp.tanh(c)

        def _acc(bk, carry):
            m, z, r = carry
            blk = x_ref[pl.ds(bk * BLK, BLK), :]
            e = lax.dot_general(blk, h, (((1,), (1,)), ((), ())),
                                preferred_element_type=jnp.float32)
            m_new = jnp.maximum(m, jnp.max(e))
            scale = jnp.exp(m - m_new)
            pp = jnp.exp(e - m_new)
            z = z * scale + jnp.sum(pp)
            r = r * scale + lax.dot_general(pp, blk, (((0,), (0,)), ((), ())),
                                            preferred_element_type=jnp.float32)
            return m_new, z, r
        _, z, r = lax.fori_loop(
            0, NBLK, _acc,
            (jnp.float32(-1e30), jnp.float32(0.0),
             jnp.zeros((1, D), jnp.float32)))
        readout = r / z
        q_star = jnp.concatenate([h, readout], axis=1)
    return q_star


def _post_body(sa_ref, sb_ref, deg_ref, feat_ref, wna_ref, wnb_ref, bn_ref,
               wnih_ref, wnhh_ref, bnih_ref, bnhh_ref,
               weih_ref, wehh_ref, beih_ref, behh_ref,
               w1_ref, b1_ref, w2_ref, b2_ref, w3_ref, b3_ref,
               out_ref, ne_ref, mean_ref):
    wna = wna_ref[...]
    wnb = wnb_ref[...]
    bn = bn_ref[...]

    def _blk(bk, carry):
        rows = (sa_ref[pl.ds(bk * BLK, BLK), :]
                + sb_ref[pl.ds(bk * BLK, BLK), :])
        deg = deg_ref[pl.ds(bk * BLK, BLK), :]
        denom = jnp.maximum(deg, 1.0)
        mean = rows / denom
        mean_ref[pl.ds(bk * BLK, BLK), :] = mean
        fb = feat_ref[pl.ds(bk * BLK, BLK), :]
        ne = jnp.dot(fb, wna, preferred_element_type=jnp.float32)
        ne = ne + jnp.dot(mean, wnb, preferred_element_type=jnp.float32)
        ne_ref[pl.ds(bk * BLK, BLK), :] = jnp.maximum(ne + bn, 0.0)
        return carry
    lax.fori_loop(0, NBLK, _blk, 0)

    node_s2s = _set2set(ne_ref, wnih_ref[...], wnhh_ref[...],
                        bnih_ref[...], bnhh_ref[...])
    edge_s2s = _set2set(mean_ref, weih_ref[...], wehh_ref[...],
                        beih_ref[...], behh_ref[...])

    out = jnp.concatenate([node_s2s, edge_s2s], axis=1)
    out = jnp.maximum(
        jnp.dot(out, w1_ref[...], preferred_element_type=jnp.float32)
        + b1_ref[...], 0.0)
    out = jnp.maximum(
        jnp.dot(out, w2_ref[...], preferred_element_type=jnp.float32)
        + b2_ref[...], 0.0)
    out_ref[...] = (
        jnp.dot(out, w3_ref[...], preferred_element_type=jnp.float32)
        + b3_ref[...])


def _post(sa, sb, deg, feat, wna, wnb, bn, wnih, wnhh, bnih, bnhh,
          weih, wehh, beih, behh, w1, b1, w2, b2, w3, b3):
    return pl.pallas_call(
        _post_body,
        out_shape=jax.ShapeDtypeStruct((1, 16), jnp.float32),
        scratch_shapes=[
            pltpu.VMEM((N, D), jnp.float32),
            pltpu.VMEM((N, D), jnp.float32),
        ],
    )(sa, sb, deg, feat, wna, wnb, bn, wnih, wnhh, bnih, bnhh,
      weih, wehh, beih, behh, w1, b1, w2, b2, w3, b3)


def kernel(feat, edge_index, efeat, W_edge, b_edge, W_node, b_node,
           Wn_ih, Wn_hh, bn_ih, bn_hh, We_ih, We_hh, be_ih, be_hh,
           W1, b1, W2, b2, W3, b3):
    a, b = _edge_tables_ab(feat, W_edge[:D], W_edge[D:2 * D])

    pad = E_PAD - E
    efeat_p = jnp.concatenate(
        [efeat, jnp.zeros((pad, DE), efeat.dtype)], axis=0)
    c = _edge_table_c(efeat_p, W_edge[2 * D:], b_edge.reshape(1, D))

    src_p = jnp.concatenate([edge_index[0], jnp.zeros((pad,), jnp.int32)])
    dst_p = jnp.concatenate([edge_index[1], jnp.full((pad,), SPAD, jnp.int32)])
    sa, dega = _sc_segment(
        a, b, c, src_p[:E_CALL].reshape(NS, NCHUNK, CH),
        dst_p[:E_CALL].reshape(NS, NCHUNK, CH), 0)
    sb, degb = _sc_segment(
        a, b, c, src_p[E_CALL:].reshape(NS, NCHUNK, CH),
        dst_p[E_CALL:].reshape(NS, NCHUNK, CH), E_CALL)

    deg_col = _deg_transpose(jnp.concatenate([dega, degb], axis=0))[:N]
    return _post(
        sa, sb, deg_col, feat,
        W_node[:D], W_node[D:], b_node.reshape(1, D),
        Wn_ih, Wn_hh, bn_ih.reshape(1, 4 * D), bn_hh.reshape(1, 4 * D),
        We_ih, We_hh, be_ih.reshape(1, 4 * D), be_hh.reshape(1, 4 * D),
        W1, b1.reshape(1, 32), W2, b2.reshape(1, 16), W3, b3.reshape(1, 16))


# speedup vs baseline: 2.1537x; 1.0271x over previous
import jax
import jax.numpy as jnp
from jax import lax
from jax.experimental import pallas as pl
from jax.experimental.pallas import tpu as pltpu
from jax.experimental.pallas import tpu_sc as plsc

N = 10000
E = 320000
D = 128
DE = 16
NS = 16
NP = 10240
SROWS = 10112
SPAD = SROWS - 1
CH = 16
GCH = 64
GROUPS = 10
NCHUNK = GROUPS * GCH
PER_TILE = NCHUNK * CH
NSPLIT = 2
E_CALL = NS * PER_TILE
E_PAD = NSPLIT * E_CALL
ROWS_PER_TILE = 624
ROWS_REM = N - NS * ROWS_PER_TILE
ZROWS = SROWS // NS
DEG_COLS = NP // NS
NBLK = 10
BLK = N // NBLK
EBLK = 4096


def _ab_body(feat_ref, ws_ref, wd_ref, a_ref, b_ref):
    f = feat_ref[...]
    a_ref[...] = jnp.dot(f, ws_ref[...], preferred_element_type=jnp.float32)
    b_ref[...] = jnp.dot(f, wd_ref[...], preferred_element_type=jnp.float32)


def _edge_tables_ab(feat, ws, wd):
    return pl.pallas_call(
        _ab_body,
        grid=(NBLK,),
        in_specs=[
            pl.BlockSpec((BLK, D), lambda i: (i, 0)),
            pl.BlockSpec((D, D), lambda i: (0, 0)),
            pl.BlockSpec((D, D), lambda i: (0, 0)),
        ],
        out_specs=[
            pl.BlockSpec((BLK, D), lambda i: (i, 0)),
            pl.BlockSpec((BLK, D), lambda i: (i, 0)),
        ],
        out_shape=[
            jax.ShapeDtypeStruct((N, D), jnp.float32),
            jax.ShapeDtypeStruct((N, D), jnp.float32),
        ],
    )(feat, ws, wd)


def _c_body(ef_ref, we_ref, be_ref, c_ref):
    c_ref[...] = (
        jnp.dot(ef_ref[...], we_ref[...], preferred_element_type=jnp.float32)
        + be_ref[...]
    )


def _edge_table_c(efeat_p, we, be):
    return pl.pallas_call(
        _c_body,
        grid=(E_PAD // EBLK,),
        in_specs=[
            pl.BlockSpec((EBLK, DE), lambda i: (i, 0)),
            pl.BlockSpec((DE, D), lambda i: (0, 0)),
            pl.BlockSpec((1, D), lambda i: (0, 0)),
        ],
        out_specs=pl.BlockSpec((EBLK, D), lambda i: (i, 0)),
        out_shape=jax.ShapeDtypeStruct((E_PAD, D), jnp.float32),
    )(efeat_p, we, be)


def _make_sc_body(base):
    def _sc_body(a_hbm, b_hbm, c_hbm, src_hbm, dst_hbm, s_out, deg_out,
                 idxs_v, idxd_v, a0_v, a1_v, b0_v, b1_v, c0_v, c1_v, r0_v, r1_v,
                 deg_v, s_sh,
                 sem_g0, sem_g1, sem_s0, sem_s1):
        sid = lax.axis_index("s")
        wid = sid
        a_v = (a0_v, a1_v)
        b_v = (b0_v, b1_v)
        c_v = (c0_v, c1_v)
        r_v = (r0_v, r1_v)
        sem_g = (sem_g0, sem_g1)
        sem_s = (sem_s0, sem_s1)

        def _zero_row(i, carry):
            for s in range(D // 16):
                r0_v[i, pl.ds(s * 16, 16)] = jnp.zeros((16,), jnp.float32)
            return carry
        lax.fori_loop(0, CH, _zero_row, 0)

        def _zero_acc(k, carry):
            pltpu.sync_copy(r0_v, s_sh.at[pl.ds(sid * ZROWS + k * CH, CH)])
            return carry
        lax.fori_loop(0, ZROWS // CH, _zero_acc, 0)
        zrem = ZROWS - (ZROWS // CH) * CH
        pltpu.sync_copy(r0_v.at[pl.ds(0, zrem)],
                        s_sh.at[pl.ds(sid * ZROWS + ZROWS - zrem, zrem)])

        def _zero_deg(i, carry):
            deg_v[pl.ds(i * 16, 16)] = jnp.zeros((16,), jnp.float32)
            return carry
        lax.fori_loop(0, NP // 16, _zero_deg, 0)

        plsc.subcore_barrier()

        ones16 = jnp.ones((16,), jnp.float32)

        def _group(g, carry):
            pltpu.sync_copy(src_hbm.at[wid, pl.ds(g * GCH, GCH)], idxs_v)
            pltpu.sync_copy(dst_hbm.at[wid, pl.ds(g * GCH, GCH)], idxd_v)
            page = 0

            def _pair(tt, c2):
                for pth in range(2):
                    li_c = 2 * (tt - 1) + pth
                    li_i = 2 * tt + pth

                    @pl.when(tt > 0)
                    def _compute():
                        @pl.when(jnp.logical_or(tt > 1, g > 0))
                        def _drain_scatter():
                            pltpu.make_async_copy(r_v[pth],
                                                  s_sh.at[pl.ds(0, CH)],
                                                  sem_s[pth]).wait()
                        def _drain_gathers(k, c4):
                            pltpu.make_async_copy(a_hbm.at[pl.ds(0, CH)],
                                                  a_v[pth], sem_g[pth]).wait()
                            return c4
                        lax.fori_loop(0, 3, _drain_gathers, 0)

                        idx16 = idxd_v[page + li_c, pl.ds(0, 16)]
                        plsc.addupdate_scatter(deg_v, [idx16], ones16)

                        def _row(i, c3):
                            for s in range(D // 16):
                                sl = pl.ds(s * 16, 16)
                                r_v[pth][i, sl] = jnp.maximum(
                                    a_v[pth][i, sl] + b_v[pth][i, sl]
                                    + c_v[pth][i, sl], 0.0)
                            return c3
                        lax.fori_loop(0, CH, _row, 0)

                        pltpu.async_copy(r_v[pth],
                                         s_sh.at[idxd_v.at[page + li_c]],
                                         sem_s[pth], add=True)

                    @pl.when(li_i < GCH)
                    def _issue():
                        pltpu.async_copy(a_hbm.at[idxs_v.at[page + li_i]],
                                         a_v[pth], sem_g[pth])
                        pltpu.async_copy(b_hbm.at[idxd_v.at[page + li_i]],
                                         b_v[pth], sem_g[pth])
                        pltpu.async_copy(
                            c_hbm.at[pl.ds(
                                base + wid * PER_TILE
                                + (g * GCH + li_i) * CH, CH)],
                            c_v[pth], sem_g[pth])
                return c2

            lax.fori_loop(0, GCH // 2 + 1, _pair, 0)
            return carry

        lax.fori_loop(0, GROUPS, _group, 0)

        for pth in range(2):
            pltpu.make_async_copy(r_v[pth], s_sh.at[pl.ds(0, CH)],
                                  sem_s[pth]).wait()

        pltpu.sync_copy(deg_v, deg_out.at[sid])

        plsc.subcore_barrier()

        pltpu.sync_copy(s_sh.at[pl.ds(sid * ROWS_PER_TILE, ROWS_PER_TILE)],
                        s_out.at[pl.ds(sid * ROWS_PER_TILE, ROWS_PER_TILE)])

        @pl.when(sid == 0)
        def _out_tail():
            pltpu.sync_copy(s_sh.at[pl.ds(NS * ROWS_PER_TILE, ROWS_REM)],
                            s_out.at[pl.ds(NS * ROWS_PER_TILE, ROWS_REM)])


    return _sc_body


def _sc_segment(a, b, c, src_r, dst_r, base):
    mesh = plsc.VectorSubcoreMesh(core_axis_name="c", subcore_axis_name="s",
                                  num_cores=1, num_subcores=NS)
    f = pl.kernel(
        _make_sc_body(base),
        mesh=mesh,
        out_type=[
            pltpu.HBM((N, D), jnp.float32),
            pltpu.HBM((NS, NP), jnp.float32),
        ],
        scratch_types=[
            pltpu.VMEM((GCH, CH), jnp.int32),
            pltpu.VMEM((GCH, CH), jnp.int32),
            pltpu.VMEM((CH, D), jnp.float32),
            pltpu.VMEM((CH, D), jnp.float32),
            pltpu.VMEM((CH, D), jnp.float32),
            pltpu.VMEM((CH, D), jnp.float32),
            pltpu.VMEM((CH, D), jnp.float32),
            pltpu.VMEM((CH, D), jnp.float32),
            pltpu.VMEM((CH, D), jnp.float32),
            pltpu.VMEM((CH, D), jnp.float32),
            pltpu.VMEM((NP,), jnp.float32),
            pltpu.VMEM_SHARED((SROWS, D), jnp.float32),
            pltpu.SemaphoreType.DMA,
            pltpu.SemaphoreType.DMA,
            pltpu.SemaphoreType.DMA,
            pltpu.SemaphoreType.DMA,
        ],
        compiler_params=pltpu.CompilerParams(needs_layout_passes=False),
    )
    return f(a, b, c, src_r, dst_r)


def _degt_body(degp_ref, out_ref):
    tot = lax.dot_general(jnp.ones((1, 2 * NS), jnp.float32), degp_ref[...],
                          (((1,), (0,)), ((), ())),
                          preferred_element_type=jnp.float32)
    ri = lax.broadcasted_iota(jnp.int32, (128, 128), 0)
    ci = lax.broadcasted_iota(jnp.int32, (128, 128), 1)
    ident = (ri == ci).astype(jnp.float32)
    for s in range(8):
        out_ref[pl.ds(s * 128, 128), :] = lax.dot_general(
            ident, tot[:, s * 128:(s + 1) * 128], (((1,), (1,)), ((), ())),
            preferred_element_type=jnp.float32)


def _deg_transpose(deg_p):
    return pl.pallas_call(
        _degt_body,
        grid=(NP // 1024,),
        in_specs=[pl.BlockSpec((2 * NS, 1024), lambda i: (0, i))],
        out_specs=pl.BlockSpec((1024, 1), lambda i: (i, 0)),
        out_shape=jax.ShapeDtypeStruct((NP, 1), jnp.float32),
    )(deg_p)


def _set2set(x_ref, w_ih, w_hh, b_ih, b_hh):
    dn = (((1,), (1,)), ((), ()))
    q_star = jnp.zeros((1, 2 * D), jnp.float32)
    h = jnp.zeros((1, D), jnp.float32)
    c = jnp.zeros((1, D), jnp.float32)
    for _ in range(3):
        gates = (lax.dot_general(q_star, w_ih, dn,
                                 preferred_element_type=jnp.float32)
                 + b_ih
                 + lax.dot_general(h, w_hh, dn,
                                   preferred_element_type=jnp.float32)
                 + b_hh)
        gi = gates[:, 0 * D:1 * D]
        gf = gates[:, 1 * D:2 * D]
        gg = gates[:, 2 * D:3 * D]
        go = gates[:, 3 * D:4 * D]
        c = jax.nn.sigmoid(gf) * c + jax.nn.sigmoid(gi) * jnp.tanh(gg)
        h = jax.nn.sigmoid(go) * jnp.tanh(c)

        def _acc(bk, carry):
            m, z, r = carry
            blk = x_ref[pl.ds(bk * BLK, BLK), :]
            e = lax.dot_general(blk, h, (((1,), (1,)), ((), ())),
                                preferred_element_type=jnp.float32)
            m_new = jnp.maximum(m, jnp.max(e))
            scale = jnp.exp(m - m_new)
            pp = jnp.exp(e - m_new)
            z = z * scale + jnp.sum(pp)
            r = r * scale + lax.dot_general(pp, blk, (((0,), (0,)), ((), ())),
                                            preferred_element_type=jnp.float32)
            return m_new, z, r
        _, z, r = lax.fori_loop(
            0, NBLK, _acc,
            (jnp.float32(-1e30), jnp.float32(0.0),
             jnp.zeros((1, D), jnp.float32)))
        readout = r / z
        q_star = jnp.concatenate([h, readout], axis=1)
    return q_star


def _post_body(sa_ref, sb_ref, deg_ref, feat_ref, wna_ref, wnb_ref, bn_ref,
               wnih_ref, wnhh_ref, bnih_ref, bnhh_ref,
               weih_ref, wehh_ref, beih_ref, behh_ref,
               w1_ref, b1_ref, w2_ref, b2_ref, w3_ref, b3_ref,
               out_ref, ne_ref, mean_ref):
    wna = wna_ref[...]
    wnb = wnb_ref[...]
    bn = bn_ref[...]

    def _blk(bk, carry):
        rows = (sa_ref[pl.ds(bk * BLK, BLK), :]
                + sb_ref[pl.ds(bk * BLK, BLK), :])
        deg = deg_ref[pl.ds(bk * BLK, BLK), :]
        denom = jnp.maximum(deg, 1.0)
        mean = rows / denom
        mean_ref[pl.ds(bk * BLK, BLK), :] = mean
        fb = feat_ref[pl.ds(bk * BLK, BLK), :]
        ne = jnp.dot(fb, wna, preferred_element_type=jnp.float32)
        ne = ne + jnp.dot(mean, wnb, preferred_element_type=jnp.float32)
        ne_ref[pl.ds(bk * BLK, BLK), :] = jnp.maximum(ne + bn, 0.0)
        return carry
    lax.fori_loop(0, NBLK, _blk, 0)

    node_s2s = _set2set(ne_ref, wnih_ref[...], wnhh_ref[...],
                        bnih_ref[...], bnhh_ref[...])
    edge_s2s = _set2set(mean_ref, weih_ref[...], wehh_ref[...],
                        beih_ref[...], behh_ref[...])

    out = jnp.concatenate([node_s2s, edge_s2s], axis=1)
    out = jnp.maximum(
        jnp.dot(out, w1_ref[...], preferred_element_type=jnp.float32)
        + b1_ref[...], 0.0)
    out = jnp.maximum(
        jnp.dot(out, w2_ref[...], preferred_element_type=jnp.float32)
        + b2_ref[...], 0.0)
    out_ref[...] = (
        jnp.dot(out, w3_ref[...], preferred_element_type=jnp.float32)
        + b3_ref[...])


def _post(sa, sb, deg, feat, wna, wnb, bn, wnih, wnhh, bnih, bnhh,
          weih, wehh, beih, behh, w1, b1, w2, b2, w3, b3):
    return pl.pallas_call(
        _post_body,
        out_shape=jax.ShapeDtypeStruct((1, 16), jnp.float32),
        scratch_shapes=[
            pltpu.VMEM((N, D), jnp.float32),
            pltpu.VMEM((N, D), jnp.float32),
        ],
    )(sa, sb, deg, feat, wna, wnb, bn, wnih, wnhh, bnih, bnhh,
      weih, wehh, beih, behh, w1, b1, w2, b2, w3, b3)


def kernel(feat, edge_index, efeat, W_edge, b_edge, W_node, b_node,
           Wn_ih, Wn_hh, bn_ih, bn_hh, We_ih, We_hh, be_ih, be_hh,
           W1, b1, W2, b2, W3, b3):
    a, b = _edge_tables_ab(feat, W_edge[:D], W_edge[D:2 * D])

    pad = E_PAD - E
    efeat_p = jnp.concatenate(
        [efeat, jnp.zeros((pad, DE), efeat.dtype)], axis=0)
    c = _edge_table_c(efeat_p, W_edge[2 * D:], b_edge.reshape(1, D))

    src_p = jnp.concatenate([edge_index[0], jnp.zeros((pad,), jnp.int32)])
    dst_p = jnp.concatenate([edge_index[1], jnp.full((pad,), SPAD, jnp.int32)])
    sa, dega = _sc_segment(
        a, b, c, src_p[:E_CALL].reshape(NS, NCHUNK, CH),
        dst_p[:E_CALL].reshape(NS, NCHUNK, CH), 0)
    sb, degb = _sc_segment(
        a, b, c, src_p[E_CALL:].reshape(NS, NCHUNK, CH),
        dst_p[E_CALL:].reshape(NS, NCHUNK, CH), E_CALL)

    deg_col = _deg_transpose(jnp.concatenate([dega, degb], axis=0))[:N]
    return _post(
        sa, sb, deg_col, feat,
        W_node[:D], W_node[D:], b_node.reshape(1, D),
        Wn_ih, Wn_hh, bn_ih.reshape(1, 4 * D), bn_hh.reshape(1, 4 * D),
        We_ih, We_hh, be_ih.reshape(1, 4 * D), be_hh.reshape(1, 4 * D),
        W1, b1.reshape(1, 32), W2, b2.reshape(1, 16), W3, b3.reshape(1, 16))
